# Initial kernel scaffold; baseline (speedup 1.0000x reference)
import jax
import jax.numpy as jnp
from jax.experimental import pallas as pl


def kernel(x, edge_index, batch, W_rel1, W_root1, b1, p1, W_rel2, W_root2, b2, p2, W_rel3, W_root3, b3, p3, Wl1, bl1, Wl2, bl2, Wl3, bl3):
    raise NotImplementedError("write your pallas kernel here")


# speedup vs baseline: 7.8142x; 7.8142x over previous
import functools
import jax
import jax.numpy as jnp
from jax import lax
from jax.experimental import pallas as pl
from jax.experimental.pallas import tpu as pltpu
from jax.experimental.pallas import tpu_sc as plsc

N = 10000
NP = 10240
E = 320000
H = 128
NG = 2
NW = 32
NCH = 80
CH = 128
EPT = NCH * CH
EPAD = NW * EPT
RPS = NP // 16

_mesh = plsc.VectorSubcoreMesh(core_axis_name="c", subcore_axis_name="s")


def _sc_agg_body(h_hbm, src_hbm, dst_hbm, zblk_hbm, out_hbm,
                 src_v, dst_v, rows_v, agg_sp, sem):
    c = lax.axis_index("c")
    s = lax.axis_index("s")
    for t in range(RPS // CH):
        pltpu.sync_copy(zblk_hbm, agg_sp.at[pl.ds(s * RPS + t * CH, CH)])
    plsc.subcore_barrier()
    wid = c * 16 + s
    pltpu.sync_copy(src_hbm.at[wid], src_v)
    pltpu.sync_copy(dst_hbm.at[wid], dst_v)

    def chunk(j, carry):
        pltpu.async_copy(h_hbm.at[src_v.at[j]], rows_v, sem).wait()
        pltpu.sync_copy(rows_v, agg_sp.at[dst_v.at[j]], add=True)
        return carry

    lax.fori_loop(0, NCH, chunk, 0)
    plsc.subcore_barrier()
    pltpu.sync_copy(agg_sp.at[pl.ds(s * RPS, RPS)],
                    out_hbm.at[c, pl.ds(s * RPS, RPS)])


_sc_agg = pl.kernel(
    _sc_agg_body,
    out_type=jax.ShapeDtypeStruct((2, NP, H), jnp.float32),
    mesh=_mesh,
    scratch_types=[
        pltpu.VMEM((NCH, CH), jnp.int32),
        pltpu.VMEM((NCH, CH), jnp.int32),
        pltpu.VMEM((CH, H), jnp.float32),
        pltpu.VMEM_SHARED((NP, H), jnp.float32),
        pltpu.SemaphoreType.DMA,
    ],
)


def _conv_body(agg_ref, h_ref, wrel_ref, wroot_ref, b_ref, p_ref, alive_ref,
               out_ref, s_ref):
    agg = agg_ref[0] + agg_ref[1]
    o = jnp.dot(agg, wrel_ref[...], preferred_element_type=jnp.float32)
    o = o + jnp.dot(h_ref[...], wroot_ref[...],
                    preferred_element_type=jnp.float32)
    o = (o + b_ref[...]) * alive_ref[...]
    o = jnp.maximum(o, 0.0)
    out_ref[...] = o
    s_ref[...] = jnp.sum(o * p_ref[...], axis=1, keepdims=True)


_BM = 1280


def _tc_conv(agg, h, wrel, wroot, b, p, alive):
    return pl.pallas_call(
        _conv_body,
        grid=(NP // _BM,),
        in_specs=[
            pl.BlockSpec((2, _BM, H), lambda i: (0, i, 0)),
            pl.BlockSpec((_BM, H), lambda i: (i, 0)),
            pl.BlockSpec((H, H), lambda i: (0, 0)),
            pl.BlockSpec((H, H), lambda i: (0, 0)),
            pl.BlockSpec((1, H), lambda i: (0, 0)),
            pl.BlockSpec((1, H), lambda i: (0, 0)),
            pl.BlockSpec((_BM, 1), lambda i: (i, 0)),
        ],
        out_specs=[
            pl.BlockSpec((_BM, H), lambda i: (i, 0)),
            pl.BlockSpec((_BM, 1), lambda i: (i, 0)),
        ],
        out_shape=[
            jax.ShapeDtypeStruct((NP, H), jnp.float32),
            jax.ShapeDtypeStruct((NP, 1), jnp.float32),
        ],
    )(agg, h, wrel, wroot, b, p, alive)


_VR = NP // 128


def _pool_body(h_ref, s_ref, alive_ref, batch_ref, p_ref,
               hp_ref, keep_ref, ro_ref):
    pvec = p_ref[...]
    norm = jnp.sqrt(jnp.sum(pvec * pvec))
    score = jnp.tanh(s_ref[...] / norm)
    batch = batch_ref[...]
    key = batch.astype(jnp.float32) * 10.0 - score
    kb = lax.bitcast_convert_type(key, jnp.int32)
    r = jnp.where(kb >= 0, kb, kb ^ jnp.int32(0x7FFFFFFF))
    aliveb = alive_ref[...] > 0.5
    idx = (lax.broadcasted_iota(jnp.int32, (_VR, 128), 0) * 128
           + lax.broadcasted_iota(jnp.int32, (_VR, 128), 1))
    IMIN = jnp.int32(-2147483648)

    def count_lt(mask, bound):
        return jnp.sum(jnp.where(mask & (r < bound), 1.0, 0.0))

    keep = jnp.zeros((_VR, 128), jnp.bool_)
    for g in range(NG):
        mg = aliveb & (batch == g)
        n_alive = jnp.sum(jnp.where(mg, 1.0, 0.0))
        k = jnp.ceil(jnp.float32(0.8) * n_alive)

        def sel_step(i, t):
            bit = 31 - i
            cand = t | (jnp.int32(1) << bit)
            c = count_lt(mg, cand ^ IMIN)
            return jnp.where(c < k, cand, t)

        tq = lax.fori_loop(0, 32, sel_step, jnp.int32(0))
        t = tq ^ IMIN
        c_lt = count_lt(mg, t)
        m_ties = k - c_lt

        def tie_step(_, lohi):
            lo, hi = lohi
            mid = (lo + hi) // 2
            c = jnp.sum(jnp.where(mg & (r == t) & (idx < mid), 1.0, 0.0))
            ok = c >= m_ties
            return jnp.where(ok, lo, mid + 1), jnp.where(ok, mid, hi)

        _, jstar = lax.fori_loop(0, 15, tie_step,
                                 (jnp.int32(0), jnp.int32(NP)))
        kg = mg & ((r < t) | ((r == t) & (idx < jstar)))
        keep = keep | kg

    keepf = keep.astype(jnp.float32)
    keep_ref[...] = keepf
    scale = score * keepf
    m0f = jnp.where(keep & (batch == 0), 1.0, 0.0)
    m1f = jnp.where(keep & (batch == 1), 1.0, 0.0)
    cnt0 = jnp.sum(m0f)
    cnt1 = jnp.sum(m1f)

    eye = (lax.broadcasted_iota(jnp.int32, (128, 128), 0)
           == lax.broadcasted_iota(jnp.int32, (128, 128), 1))

    def col(vec_row):
        return jnp.sum(jnp.where(eye, vec_row, 0.0), axis=1, keepdims=True)

    mx0 = jnp.full((1, H), -1e9, jnp.float32)
    mx1 = jnp.full((1, H), -1e9, jnp.float32)
    sm0 = jnp.zeros((1, H), jnp.float32)
    sm1 = jnp.zeros((1, H), jnp.float32)
    for j in range(_VR):
        hb = h_ref[pl.ds(j * 128, 128), :] * col(scale[j:j + 1, :])
        hp_ref[pl.ds(j * 128, 128), :] = hb
        m0c = col(m0f[j:j + 1, :]) > 0.5
        m1c = col(m1f[j:j + 1, :]) > 0.5
        mx0 = jnp.maximum(mx0, jnp.max(jnp.where(m0c, hb, -1e9), axis=0,
                                       keepdims=True))
        mx1 = jnp.maximum(mx1, jnp.max(jnp.where(m1c, hb, -1e9), axis=0,
                                       keepdims=True))
        sm0 = sm0 + jnp.sum(jnp.where(m0c, hb, 0.0), axis=0, keepdims=True)
        sm1 = sm1 + jnp.sum(jnp.where(m1c, hb, 0.0), axis=0, keepdims=True)

    ro = jnp.concatenate([
        jnp.concatenate([mx0, sm0 / cnt0], axis=1),
        jnp.concatenate([mx1, sm1 / cnt1], axis=1),
        jnp.zeros((6, 2 * H), jnp.float32),
    ], axis=0)
    ro_ref[...] = ro


def _tc_pool(h, s, alive, batch, p):
    return pl.pallas_call(
        _pool_body,
        out_shape=[
            jax.ShapeDtypeStruct((NP, H), jnp.float32),
            jax.ShapeDtypeStruct((_VR, 128), jnp.float32),
            jax.ShapeDtypeStruct((8, 2 * H), jnp.float32),
        ],
    )(h, s, alive, batch, p)


def _mlp_body(r1_ref, r2_ref, r3_ref, w1_ref, b1_ref, w2_ref, b2_ref,
              w3_ref, b3_ref, out_ref):
    z = r1_ref[...] + r2_ref[...] + r3_ref[...]
    z = jnp.maximum(jnp.dot(z, w1_ref[...],
                            preferred_element_type=jnp.float32)
                    + b1_ref[...], 0.0)
    z = jnp.maximum(jnp.dot(z, w2_ref[...],
                            preferred_element_type=jnp.float32)
                    + b2_ref[...], 0.0)
    l = jnp.dot(z, w3_ref[...], preferred_element_type=jnp.float32) \
        + b3_ref[...]
    col = lax.broadcasted_iota(jnp.int32, l.shape, 1)
    l = jnp.where(col < 15, l, -1e30)
    mx = jnp.max(l, axis=1, keepdims=True)
    sh = l - mx
    lse = jnp.log(jnp.sum(jnp.where(col < 15, jnp.exp(sh), 0.0), axis=1,
                          keepdims=True))
    out_ref[...] = sh - lse


def _tc_mlp(r1, r2, r3, w1, b1, w2, b2, w3, b3):
    return pl.pallas_call(
        _mlp_body,
        out_shape=jax.ShapeDtypeStruct((8, H), jnp.float32),
    )(r1, r2, r3, w1, b1, w2, b2, w3, b3)


def kernel(x, edge_index, batch, W_rel1, W_root1, b1, p1, W_rel2, W_root2,
           b2, p2, W_rel3, W_root3, b3, p3, Wl1, bl1, Wl2, bl2, Wl3, bl3):
    src = edge_index[0]
    dst = edge_index[1]
    srcp = jnp.concatenate(
        [src, jnp.zeros((EPAD - E,), jnp.int32)]).reshape(NW, NCH, CH)
    dstp = jnp.concatenate(
        [dst, jnp.full((EPAD - E,), N, jnp.int32)]).reshape(NW, NCH, CH)
    xp = jnp.pad(x, ((0, NP - N), (0, 0)))
    batchp = jnp.concatenate(
        [batch, jnp.full((NP - N,), NG + 1, jnp.int32)]).reshape(_VR, 128)
    alive = jnp.concatenate(
        [jnp.ones((N,), jnp.float32),
         jnp.zeros((NP - N,), jnp.float32)]).reshape(_VR, 128)
    zblk = jnp.zeros((CH, H), jnp.float32)

    h = xp
    ros = []
    for (wrel, wroot, b, p) in ((W_rel1, W_root1, b1, p1),
                                (W_rel2, W_root2, b2, p2),
                                (W_rel3, W_root3, b3, p3)):
        agg = _sc_agg(h, srcp, dstp, zblk)
        h_new, s = _tc_conv(agg, h, wrel, wroot, b.reshape(1, H),
                            p.reshape(1, H), alive.reshape(NP, 1))
        h, alive, ro = _tc_pool(h_new, s.reshape(_VR, 128), alive, batchp,
                                p.reshape(1, H))
        ros.append(ro)

    w2p = jnp.pad(Wl2, ((0, 0), (0, H - 64)))
    b2p = jnp.pad(bl2, (0, H - 64)).reshape(1, H)
    w3p = jnp.pad(Wl3, ((0, H - 64), (0, H - 15)))
    b3p = jnp.pad(bl3, (0, H - 15)).reshape(1, H)
    out = _tc_mlp(ros[0], ros[1], ros[2], Wl1, bl1.reshape(1, H),
                  w2p, b2p, w3p, b3p)
    return out[:2, :15]
